# Initial kernel scaffold; baseline (speedup 1.0000x reference)
import jax
import jax.numpy as jnp
from jax.experimental import pallas as pl


def kernel(x, edge_index, gin_eps, W1, b1, g1, be1, W2, b2, g2, be2):
    raise NotImplementedError("write your pallas kernel here")


# speedup vs baseline: 3.3973x; 3.3973x over previous
import functools

import jax
import jax.numpy as jnp
from jax import lax
from jax.experimental import pallas as pl
from jax.experimental.pallas import tpu as pltpu
from jax.experimental.pallas import tpu_sc as plsc

N = 10000
E = 320000
D = 128

NC = 2
NS = 16
NW = NC * NS
CHUNK = 128
EROWS = -(-E // (CHUNK * NW * 8)) * NW * 8
E_PAD = EROWS * CHUNK
CPW = EROWS // NW
RPT = 640
N_PAD = NS * RPT
ZR = 128


def _agg_body(x_hbm, src_hbm, dst_hbm, out_hbm,
              src_v, dst_v, rows_v, acc, gsem):
    c = lax.axis_index("c")
    s = lax.axis_index("s")
    wid = c * NS + s

    def zrow(i, carry):
        for k in range(D // 16):
            rows_v[i, pl.ds(k * 16, 16)] = jnp.zeros((16,), jnp.float32)
        return carry

    lax.fori_loop(0, ZR, zrow, 0)
    for r in range(RPT // ZR):
        pltpu.sync_copy(rows_v, acc.at[pl.ds(s * RPT + r * ZR, ZR)])
    plsc.subcore_barrier()

    erow = wid * CPW
    pltpu.sync_copy(src_hbm.at[pl.ds(erow, CPW), :], src_v)
    pltpu.sync_copy(dst_hbm.at[pl.ds(erow, CPW), :], dst_v)

    def body(j, carry):
        pltpu.async_copy(x_hbm.at[src_v.at[j]], rows_v, gsem).wait()
        pltpu.sync_copy(rows_v, acc.at[dst_v.at[j]], add=True)
        return carry

    lax.fori_loop(0, CPW, body, 0)
    plsc.subcore_barrier()

    pltpu.sync_copy(acc.at[pl.ds(s * RPT, RPT)],
                    out_hbm.at[c, pl.ds(s * RPT, RPT), :])


_agg = functools.partial(
    pl.kernel,
    mesh=plsc.VectorSubcoreMesh(core_axis_name="c", subcore_axis_name="s"),
    out_type=jax.ShapeDtypeStruct((NC, N_PAD, D), jnp.float32),
    scratch_types=[
        pltpu.VMEM((CPW, CHUNK), jnp.int32),
        pltpu.VMEM((CPW, CHUNK), jnp.int32),
        pltpu.VMEM((CHUNK, D), jnp.float32),
        pltpu.VMEM_SHARED((N_PAD, D), jnp.float32),
        pltpu.SemaphoreType.DMA,
    ],
)(_agg_body)


def _mlp_body(eps_ref, x_ref, p_ref, w1_ref, b1_ref, g1_ref, be1_ref,
              w2_ref, b2_ref, g2_ref, be2_ref, o_ref):
    xv = x_ref[...]
    agg = p_ref[0, :N, :] + p_ref[1, :N, :]
    a = (1.0 + eps_ref[0]) * xv + agg
    z = jnp.dot(a, w1_ref[...], preferred_element_type=jnp.float32)
    z = z + b1_ref[...]
    m = jnp.mean(z, axis=0, keepdims=True)
    v = jnp.mean((z - m) ** 2, axis=0, keepdims=True)
    z = (z - m) / jnp.sqrt(v + 1e-5) * g1_ref[...] + be1_ref[...]
    z = jnp.maximum(z, 0.0)
    h = jnp.dot(z, w2_ref[...], preferred_element_type=jnp.float32)
    h = h + b2_ref[...]
    m2 = jnp.mean(h, axis=0, keepdims=True)
    v2 = jnp.mean((h - m2) ** 2, axis=0, keepdims=True)
    h = (h - m2) / jnp.sqrt(v2 + 1e-5) * g2_ref[...] + be2_ref[...]
    o_ref[...] = xv + jnp.maximum(h, 0.0)


_mlp = pl.pallas_call(
    _mlp_body,
    out_shape=jax.ShapeDtypeStruct((N, D), jnp.float32),
    in_specs=[pl.BlockSpec(memory_space=pltpu.SMEM)]
    + [pl.BlockSpec(memory_space=pltpu.VMEM)] * 10,
    out_specs=pl.BlockSpec(memory_space=pltpu.VMEM),
)


def kernel(x, edge_index, gin_eps, W1, b1, g1, be1, W2, b2, g2, be2):
    src = edge_index[0].astype(jnp.int32)
    dst = edge_index[1].astype(jnp.int32)
    pad = E_PAD - E
    src = jnp.concatenate([src, jnp.zeros((pad,), jnp.int32)])
    dst = jnp.concatenate([dst, jnp.full((pad,), N, jnp.int32)])
    partials = _agg(x, src.reshape(EROWS, CHUNK), dst.reshape(EROWS, CHUNK))
    return _mlp(gin_eps.reshape(1), x, partials,
                W1, b1.reshape(1, D), g1.reshape(1, D), be1.reshape(1, D),
                W2, b2.reshape(1, D), g2.reshape(1, D), be2.reshape(1, D))
